# baseline (speedup 1.0000x reference)
import jax
import jax.numpy as jnp
from jax import lax
from jax.experimental import pallas as pl
from jax.experimental.pallas import tpu as pltpu
from jax.experimental.pallas import tpu_sc as plsc

N_NODES = 50000
N_EDGES = 800000
IN_DIM = 128
HID = 64
N_CLASSES = 8
N_GRAPHS = 64

NC = 2
NS = 16
CHUNK = 128
GROUP = 3
PAIRS = 66
DEG_GROUP = 3
N_CHUNKS = NS * 2 * GROUP * PAIRS
E_PAD = N_CHUNKS * CHUNK
NPAD = 50176
HROWS = NPAD * 32 // 128
DROWS = NPAD * 16 // 128
BROWS = NPAD // 128
SUB_H = HROWS // NS
SUB_D = DROWS // NS
DEG_GROUPS = N_CHUNKS // (NC * NS * DEG_GROUP)

GRID = 7
BLKN = NPAD // GRID
BLKH = HROWS // GRID
BLKD = DROWS // GRID
BLKB = BROWS // GRID

_mesh = plsc.VectorSubcoreMesh(core_axis_name="c", subcore_axis_name="s")
_sc_params = pltpu.CompilerParams(use_tc_tiling_on_sc=False)

_HIGH = jax.lax.Precision.HIGHEST


def _dot(a, b):
  return jax.lax.dot_general(a, b, (((1,), (0,)), ((), ())),
                             precision=_HIGH,
                             preferred_element_type=jnp.float32)


def _deg_body(col_hbm, e0_hbm, zeros_hbm, deg_hbm, cbuf, valbuf, acc, sem):
  c = lax.axis_index("c")
  s = lax.axis_index("s")
  nb = pl.multiple_of(s * (NPAD // NS), 8)
  pltpu.sync_copy(zeros_hbm.at[pl.ds(nb, NPAD // NS)],
                  acc.at[pl.ds(nb, NPAD // NS)])
  pltpu.sync_copy(e0_hbm, valbuf)
  plsc.subcore_barrier()

  base0 = c * (N_CHUNKS // NC) + s * (DEG_GROUPS * DEG_GROUP)

  @pl.loop(0, DEG_GROUPS)
  def _(g):
    base = base0 + g * DEG_GROUP
    pltpu.sync_copy(col_hbm.at[pl.ds(base, DEG_GROUP)], cbuf)
    waits = []
    for j in range(DEG_GROUP):
      waits.append(pltpu.async_copy(valbuf, acc.at[cbuf.at[j]], sem, add=True))
    for w in waits:
      w.wait()

  plsc.subcore_barrier()
  pltpu.sync_copy(acc.at[pl.ds(nb, NPAD // NS)],
                  deg_hbm.at[c, pl.ds(nb, NPAD // NS)])


_deg_call = pl.kernel(
    _deg_body,
    out_type=jax.ShapeDtypeStruct((NC, NPAD, 16), jnp.float32),
    mesh=_mesh,
    scratch_types=[
        pltpu.VMEM((DEG_GROUP, CHUNK), jnp.int32),
        pltpu.VMEM((CHUNK, 16), jnp.float32),
        pltpu.VMEM_SHARED((NPAD, 16), jnp.float32),
        pltpu.SemaphoreType.DMA,
    ],
    compiler_params=_sc_params,
)


def _agg_body(hs_hbm, row_hbm, col_hbm, out_hbm,
              rbufA, cbufA, rbufB, cbufB,
              a0, a1, a2, b0, b1, b2, acc,
              sem_gA, sem_gB, sem_sA, sem_sB):
  valsA = [a0, a1, a2]
  valsB = [b0, b1, b2]
  c = lax.axis_index("c")
  s = lax.axis_index("s")
  nb = pl.multiple_of(s * (NPAD // NS), 8)
  pltpu.sync_copy(hs_hbm.at[c, pl.ds(nb, NPAD // NS)],
                  acc.at[pl.ds(nb, NPAD // NS)])
  plsc.subcore_barrier()

  base0 = s * (2 * GROUP * PAIRS)

  def drain_scatters(vals, cbuf, sem):
    for k in range(GROUP):
      pltpu.make_async_copy(vals[k], acc.at[cbuf.at[k]], sem).wait()

  @pl.loop(0, PAIRS)
  def _(i):
    baseA = base0 + i * 2 * GROUP
    baseB = baseA + GROUP

    @pl.when(i > 0)
    def _():
      drain_scatters(valsA, cbufA, sem_sA)

    pltpu.sync_copy(row_hbm.at[pl.ds(baseA, GROUP)], rbufA)
    pltpu.sync_copy(col_hbm.at[pl.ds(baseA, GROUP)], cbufA)
    gA = [pltpu.async_copy(hs_hbm.at[c].at[rbufA.at[k]], valsA[k], sem_gA)
          for k in range(GROUP)]

    @pl.when(i > 0)
    def _():
      drain_scatters(valsB, cbufB, sem_sB)

    pltpu.sync_copy(row_hbm.at[pl.ds(baseB, GROUP)], rbufB)
    pltpu.sync_copy(col_hbm.at[pl.ds(baseB, GROUP)], cbufB)
    gB = [pltpu.async_copy(hs_hbm.at[c].at[rbufB.at[k]], valsB[k], sem_gB)
          for k in range(GROUP)]

    for w in gA:
      w.wait()
    for k in range(GROUP):
      pltpu.async_copy(valsA[k], acc.at[cbufA.at[k]], sem_sA, add=True)
    for w in gB:
      w.wait()
    for k in range(GROUP):
      pltpu.async_copy(valsB[k], acc.at[cbufB.at[k]], sem_sB, add=True)

  drain_scatters(valsA, cbufA, sem_sA)
  drain_scatters(valsB, cbufB, sem_sB)

  plsc.subcore_barrier()
  pltpu.sync_copy(acc.at[pl.ds(nb, NPAD // NS)],
                  out_hbm.at[c, pl.ds(nb, NPAD // NS)])


_agg_call = pl.kernel(
    _agg_body,
    out_type=jax.ShapeDtypeStruct((NC, NPAD, HID // 2), jnp.float32),
    mesh=_mesh,
    scratch_types=(
        [pltpu.VMEM((GROUP, CHUNK), jnp.int32)] * 4
        + [pltpu.VMEM((CHUNK, HID // 2), jnp.float32)] * (2 * GROUP)
        + [pltpu.VMEM_SHARED((NPAD, HID // 2), jnp.float32)]
        + [pltpu.SemaphoreType.DMA] * 4
    ),
    compiler_params=_sc_params,
)


def _mm1_body(x_ref, w_ref, o_ref):
  o_ref[...] = _dot(x_ref[...], w_ref[...])


_mm1 = pl.pallas_call(
    _mm1_body,
    grid=(25,),
    in_specs=[
        pl.BlockSpec((2000, IN_DIM), lambda i: (i, 0)),
        pl.BlockSpec((IN_DIM, HID), lambda i: (0, 0)),
    ],
    out_specs=pl.BlockSpec((2000, HID), lambda i: (i, 0)),
    out_shape=jax.ShapeDtypeStruct((N_NODES, HID), jnp.float32),
)


def _scale1_body(h_ref, deg_ref, hs_ref, dinv_ref):
  deg = deg_ref[0] + deg_ref[1]
  tot = 1.0 + jnp.sum(deg, axis=1)
  dinv = jax.lax.rsqrt(tot)[:, None]
  i = pl.program_id(0)
  nid = i * BLKN + jax.lax.broadcasted_iota(jnp.int32, (BLKN, 1), 0)
  hs = jnp.where(nid < N_NODES, h_ref[...] * dinv, 0.0)
  hs_ref[0] = hs[:, :HID // 2]
  hs_ref[1] = hs[:, HID // 2:]
  dinv_ref[...] = jnp.broadcast_to(dinv, (BLKN, 32))


_scale1 = pl.pallas_call(
    _scale1_body,
    grid=(GRID,),
    in_specs=[
        pl.BlockSpec((BLKN, HID), lambda i: (i, 0)),
        pl.BlockSpec((NC, BLKN, 16), lambda i: (0, i, 0)),
    ],
    out_specs=[
        pl.BlockSpec((NC, BLKN, HID // 2), lambda i: (0, i, 0)),
        pl.BlockSpec((BLKN, 32), lambda i: (i, 0)),
    ],
    out_shape=[
        jax.ShapeDtypeStruct((NC, NPAD, HID // 2), jnp.float32),
        jax.ShapeDtypeStruct((NPAD, 32), jnp.float32),
    ],
)


def _layer_body(acc_ref, dinv_ref, w_ref, b_ref, hs_ref):
  d = dinv_ref[...]
  r0 = jnp.maximum(acc_ref[0] * d + b_ref[0:1, :], 0.0)
  r1 = jnp.maximum(acc_ref[1] * d + b_ref[1:2, :], 0.0)
  h0 = _dot(r0, w_ref[0, 0]) + _dot(r1, w_ref[1, 0])
  h1 = _dot(r0, w_ref[0, 1]) + _dot(r1, w_ref[1, 1])
  hs_ref[0] = h0 * d
  hs_ref[1] = h1 * d


_layer = pl.pallas_call(
    _layer_body,
    grid=(GRID,),
    in_specs=[
        pl.BlockSpec((NC, BLKH, 128), lambda i: (0, i, 0)),
        pl.BlockSpec((BLKH, 128), lambda i: (i, 0)),
        pl.BlockSpec((2, 2, 128, 128), lambda i: (0, 0, 0, 0)),
        pl.BlockSpec((2, 128), lambda i: (0, 0)),
    ],
    out_specs=pl.BlockSpec((NC, BLKH, 128), lambda i: (0, i, 0)),
    out_shape=jax.ShapeDtypeStruct((NC, HROWS, 128), jnp.float32),
)


def _pool_body(acc_ref, dinv_ref, b_ref, bjs_ref, wc_ref, bc_ref,
               out_ref, pool_scr):
  i = pl.program_id(0)

  @pl.when(i == 0)
  def _():
    pool_scr[...] = jnp.zeros_like(pool_scr)

  d = dinv_ref[...]
  r0 = jnp.maximum(acc_ref[0] * d + b_ref[0:1, :], 0.0)
  r1 = jnp.maximum(acc_ref[1] * d + b_ref[1:2, :], 0.0)
  laneblk = jax.lax.broadcasted_iota(jnp.int32, (BLKH, 128), 1) // 32
  giota = jax.lax.broadcasted_iota(jnp.int32, (BLKH, N_GRAPHS), 1)
  for j in range(4):
    mj = laneblk == j
    zc = jnp.concatenate(
        [jnp.where(mj, r0, 0.0), jnp.where(mj, r1, 0.0),
         jnp.ones((BLKH, 8), jnp.float32)], axis=1)
    ohj = (bjs_ref[:, j:j + 1] == giota).astype(jnp.float32)
    pool_scr[...] += jax.lax.dot_general(
        ohj, zc, (((0,), (0,)), ((), ())),
        precision=_HIGH, preferred_element_type=jnp.float32)

  @pl.when(i == GRID - 1)
  def _():
    pool = pool_scr[...]
    l32 = jax.lax.broadcasted_iota(jnp.int32, (128, 32), 0) % 32
    fold = (l32 == jax.lax.broadcasted_iota(jnp.int32, (128, 32), 1)
            ).astype(jnp.float32)
    sums = jnp.concatenate(
        [_dot(pool[:, :128], fold), _dot(pool[:, 128:256], fold)], axis=1)
    counts = jnp.maximum(pool[:, 256:257], 1.0)
    out_ref[...] = _dot(sums / counts, wc_ref[...]) + bc_ref[...]


_pool = pl.pallas_call(
    _pool_body,
    grid=(GRID,),
    in_specs=[
        pl.BlockSpec((NC, BLKH, 128), lambda i: (0, i, 0)),
        pl.BlockSpec((BLKH, 128), lambda i: (i, 0)),
        pl.BlockSpec((2, 128), lambda i: (0, 0)),
        pl.BlockSpec((BLKH, 4), lambda i: (i, 0)),
        pl.BlockSpec((HID, N_CLASSES), lambda i: (0, 0)),
        pl.BlockSpec((1, N_CLASSES), lambda i: (0, 0)),
    ],
    out_specs=pl.BlockSpec((N_GRAPHS, N_CLASSES), lambda i: (0, 0)),
    out_shape=jax.ShapeDtypeStruct((N_GRAPHS, N_CLASSES), jnp.float32),
    scratch_shapes=[pltpu.VMEM((N_GRAPHS, 264), jnp.float32)],
)


def _bd(W):
  eye4 = jnp.eye(4, dtype=W.dtype)
  return jnp.stack([
      jnp.stack([jnp.kron(eye4, W[i * 32:(i + 1) * 32, o * 32:(o + 1) * 32])
                 for o in range(2)])
      for i in range(2)])


def _bt(b):
  return jnp.stack([jnp.tile(b[:32], 4), jnp.tile(b[32:], 4)])


@jax.jit
def kernel(x, edge_index, batch, W1, b1, W2, b2, W3, b3, Wc, bc):
  row = edge_index[0].astype(jnp.int32)
  col = edge_index[1].astype(jnp.int32)
  npad = E_PAD - N_EDGES
  junkc = N_NODES + (jnp.arange(npad, dtype=jnp.int32) % (NPAD - N_NODES))
  row2d = jnp.concatenate([row, jnp.zeros((npad,), jnp.int32)]).reshape(
      N_CHUNKS, CHUNK)
  col2d = jnp.concatenate([col, junkc]).reshape(N_CHUNKS, CHUNK)
  bjs = jnp.concatenate(
      [batch.astype(jnp.int32),
       jnp.full((NPAD - N_NODES,), -1, jnp.int32)]).reshape(HROWS, 4)

  e0 = jnp.zeros((CHUNK, 16), jnp.float32).at[:, 0].set(1.0)
  zerosD = jnp.zeros((NPAD, 16), jnp.float32)

  def pack(a):
    return a.reshape(NC, HROWS, 128)

  def unpack(a):
    return a.reshape(NC, NPAD, HID // 2)

  deg = _deg_call(col2d, e0, zerosD)
  h1 = _mm1(x, W1)
  hs1, dinv32 = _scale1(h1, deg)
  dinvP = dinv32.reshape(HROWS, 128)
  acc1 = _agg_call(hs1, row2d, col2d)
  hs2 = _layer(pack(acc1), dinvP, _bd(W2), _bt(b1))
  acc2 = _agg_call(unpack(hs2), row2d, col2d)
  hs3 = _layer(pack(acc2), dinvP, _bd(W3), _bt(b2))
  acc3 = _agg_call(unpack(hs3), row2d, col2d)
  return _pool(pack(acc3), dinvP, _bt(b3), bjs,
               Wc, bc.reshape(1, N_CLASSES))


# speedup vs baseline: 1.1360x; 1.1360x over previous
import jax
import jax.numpy as jnp
from jax import lax
from jax.experimental import pallas as pl
from jax.experimental.pallas import tpu as pltpu
from jax.experimental.pallas import tpu_sc as plsc

N_NODES = 50000
N_EDGES = 800000
IN_DIM = 128
HID = 64
N_CLASSES = 8
N_GRAPHS = 64

NC = 2
NS = 16
CHUNK = 128
GROUP = 4
GROUPS = 98
DEG_GROUP = 4
N_CHUNKS = NS * GROUP * GROUPS
E_PAD = N_CHUNKS * CHUNK
NPAD = 50176
HROWS = NPAD * 32 // 128
DROWS = NPAD * 16 // 128
BROWS = NPAD // 128
SUB_H = HROWS // NS
SUB_D = DROWS // NS
DEG_GROUPS = N_CHUNKS // (NC * NS * DEG_GROUP)

GRID = 7
BLKN = NPAD // GRID
BLKH = HROWS // GRID
BLKD = DROWS // GRID
BLKB = BROWS // GRID

_mesh = plsc.VectorSubcoreMesh(core_axis_name="c", subcore_axis_name="s")
_sc_params = pltpu.CompilerParams(use_tc_tiling_on_sc=False)

_HIGH = jax.lax.Precision.HIGHEST


def _dot(a, b):
  return jax.lax.dot_general(a, b, (((1,), (0,)), ((), ())),
                             precision=_HIGH,
                             preferred_element_type=jnp.float32)


def _deg_body(col_hbm, e0_hbm, zeros_hbm, deg_hbm, cbuf, valbuf, acc, sem):
  c = lax.axis_index("c")
  s = lax.axis_index("s")
  nb = pl.multiple_of(s * (NPAD // NS), 8)
  pltpu.sync_copy(zeros_hbm.at[pl.ds(nb, NPAD // NS)],
                  acc.at[pl.ds(nb, NPAD // NS)])
  pltpu.sync_copy(e0_hbm, valbuf)
  plsc.subcore_barrier()

  base0 = c * (N_CHUNKS // NC) + s * (DEG_GROUPS * DEG_GROUP)

  @pl.loop(0, DEG_GROUPS)
  def _(g):
    base = base0 + g * DEG_GROUP
    pltpu.sync_copy(col_hbm.at[pl.ds(base, DEG_GROUP)], cbuf)
    waits = []
    for j in range(DEG_GROUP):
      waits.append(pltpu.async_copy(valbuf, acc.at[cbuf.at[j]], sem, add=True))
    for w in waits:
      w.wait()

  plsc.subcore_barrier()
  pltpu.sync_copy(acc.at[pl.ds(nb, NPAD // NS)],
                  deg_hbm.at[c, pl.ds(nb, NPAD // NS)])


_deg_call = pl.kernel(
    _deg_body,
    out_type=jax.ShapeDtypeStruct((NC, NPAD, 16), jnp.float32),
    mesh=_mesh,
    scratch_types=[
        pltpu.VMEM((DEG_GROUP, CHUNK), jnp.int32),
        pltpu.VMEM((CHUNK, 16), jnp.float32),
        pltpu.VMEM_SHARED((NPAD, 16), jnp.float32),
        pltpu.SemaphoreType.DMA,
    ],
    compiler_params=_sc_params,
)


def _agg_body(hs_hbm, row_hbm, col_hbm, out_hbm,
              rbuf, cbuf, v0, v1, v2, v3, acc, sem_g, sem_s):
  vals = [v0, v1, v2, v3]
  c = lax.axis_index("c")
  s = lax.axis_index("s")
  nb = pl.multiple_of(s * (NPAD // NS), 8)
  pltpu.sync_copy(hs_hbm.at[c, pl.ds(nb, NPAD // NS)],
                  acc.at[pl.ds(nb, NPAD // NS)])
  plsc.subcore_barrier()

  base0 = s * (GROUP * GROUPS)

  @pl.loop(0, GROUPS // 2)
  def _(g):
    base = base0 + g * (2 * GROUP)
    pltpu.sync_copy(row_hbm.at[pl.ds(base, 2 * GROUP)], rbuf)
    pltpu.sync_copy(col_hbm.at[pl.ds(base, 2 * GROUP)], cbuf)
    for half in range(2):
      gathers = []
      for k in range(GROUP):
        gathers.append(pltpu.async_copy(
            hs_hbm.at[c].at[rbuf.at[half * GROUP + k]], vals[k], sem_g))
      for w in gathers:
        w.wait()
      scatters = []
      for k in range(GROUP):
        scatters.append(pltpu.async_copy(
            vals[k], acc.at[cbuf.at[half * GROUP + k]], sem_s, add=True))
      for w in scatters:
        w.wait()

  plsc.subcore_barrier()
  pltpu.sync_copy(acc.at[pl.ds(nb, NPAD // NS)],
                  out_hbm.at[c, pl.ds(nb, NPAD // NS)])


_agg_call = pl.kernel(
    _agg_body,
    out_type=jax.ShapeDtypeStruct((NC, NPAD, HID // 2), jnp.float32),
    mesh=_mesh,
    scratch_types=(
        [pltpu.VMEM((2 * GROUP, CHUNK), jnp.int32)] * 2
        + [pltpu.VMEM((CHUNK, HID // 2), jnp.float32)] * GROUP
        + [pltpu.VMEM_SHARED((NPAD, HID // 2), jnp.float32)]
        + [pltpu.SemaphoreType.DMA] * 2
    ),
    compiler_params=_sc_params,
)


def _mm1_body(x_ref, w_ref, o_ref):
  o_ref[...] = _dot(x_ref[...], w_ref[...])


_mm1 = pl.pallas_call(
    _mm1_body,
    grid=(25,),
    in_specs=[
        pl.BlockSpec((2000, IN_DIM), lambda i: (i, 0)),
        pl.BlockSpec((IN_DIM, HID), lambda i: (0, 0)),
    ],
    out_specs=pl.BlockSpec((2000, HID), lambda i: (i, 0)),
    out_shape=jax.ShapeDtypeStruct((N_NODES, HID), jnp.float32),
)


def _scale1_body(h_ref, deg_ref, hs_ref, dinv_ref):
  deg = deg_ref[0] + deg_ref[1]
  tot = 1.0 + jnp.sum(deg, axis=1)
  dinv = jax.lax.rsqrt(tot)[:, None]
  i = pl.program_id(0)
  nid = i * BLKN + jax.lax.broadcasted_iota(jnp.int32, (BLKN, 1), 0)
  hs = jnp.where(nid < N_NODES, h_ref[...] * dinv, 0.0)
  hs_ref[0] = hs[:, :HID // 2]
  hs_ref[1] = hs[:, HID // 2:]
  dinv_ref[...] = jnp.broadcast_to(dinv, (BLKN, 32))


_scale1 = pl.pallas_call(
    _scale1_body,
    grid=(GRID,),
    in_specs=[
        pl.BlockSpec((BLKN, HID), lambda i: (i, 0)),
        pl.BlockSpec((NC, BLKN, 16), lambda i: (0, i, 0)),
    ],
    out_specs=[
        pl.BlockSpec((NC, BLKN, HID // 2), lambda i: (0, i, 0)),
        pl.BlockSpec((BLKN, 32), lambda i: (i, 0)),
    ],
    out_shape=[
        jax.ShapeDtypeStruct((NC, NPAD, HID // 2), jnp.float32),
        jax.ShapeDtypeStruct((NPAD, 32), jnp.float32),
    ],
)


def _layer_body(acc_ref, dinv_ref, w_ref, b_ref, hs_ref):
  d = dinv_ref[...]
  r0 = jnp.maximum(acc_ref[0] * d + b_ref[0:1, :], 0.0)
  r1 = jnp.maximum(acc_ref[1] * d + b_ref[1:2, :], 0.0)
  h0 = _dot(r0, w_ref[0, 0]) + _dot(r1, w_ref[1, 0])
  h1 = _dot(r0, w_ref[0, 1]) + _dot(r1, w_ref[1, 1])
  hs_ref[0] = h0 * d
  hs_ref[1] = h1 * d


_layer = pl.pallas_call(
    _layer_body,
    grid=(GRID,),
    in_specs=[
        pl.BlockSpec((NC, BLKH, 128), lambda i: (0, i, 0)),
        pl.BlockSpec((BLKH, 128), lambda i: (i, 0)),
        pl.BlockSpec((2, 2, 128, 128), lambda i: (0, 0, 0, 0)),
        pl.BlockSpec((2, 128), lambda i: (0, 0)),
    ],
    out_specs=pl.BlockSpec((NC, BLKH, 128), lambda i: (0, i, 0)),
    out_shape=jax.ShapeDtypeStruct((NC, HROWS, 128), jnp.float32),
)


def _pool_body(acc_ref, dinv_ref, b_ref, bjs_ref, wc_ref, bc_ref,
               out_ref, pool_scr):
  i = pl.program_id(0)

  @pl.when(i == 0)
  def _():
    pool_scr[...] = jnp.zeros_like(pool_scr)

  d = dinv_ref[...]
  r0 = jnp.maximum(acc_ref[0] * d + b_ref[0:1, :], 0.0)
  r1 = jnp.maximum(acc_ref[1] * d + b_ref[1:2, :], 0.0)
  laneblk = jax.lax.broadcasted_iota(jnp.int32, (BLKH, 128), 1) // 32
  giota = jax.lax.broadcasted_iota(jnp.int32, (BLKH, N_GRAPHS), 1)
  for j in range(4):
    mj = laneblk == j
    zc = jnp.concatenate(
        [jnp.where(mj, r0, 0.0), jnp.where(mj, r1, 0.0),
         jnp.ones((BLKH, 8), jnp.float32)], axis=1)
    ohj = (bjs_ref[:, j:j + 1] == giota).astype(jnp.float32)
    pool_scr[...] += jax.lax.dot_general(
        ohj, zc, (((0,), (0,)), ((), ())),
        precision=_HIGH, preferred_element_type=jnp.float32)

  @pl.when(i == GRID - 1)
  def _():
    pool = pool_scr[...]
    l32 = jax.lax.broadcasted_iota(jnp.int32, (128, 32), 0) % 32
    fold = (l32 == jax.lax.broadcasted_iota(jnp.int32, (128, 32), 1)
            ).astype(jnp.float32)
    sums = jnp.concatenate(
        [_dot(pool[:, :128], fold), _dot(pool[:, 128:256], fold)], axis=1)
    counts = jnp.maximum(pool[:, 256:257], 1.0)
    out_ref[...] = _dot(sums / counts, wc_ref[...]) + bc_ref[...]


_pool = pl.pallas_call(
    _pool_body,
    grid=(GRID,),
    in_specs=[
        pl.BlockSpec((NC, BLKH, 128), lambda i: (0, i, 0)),
        pl.BlockSpec((BLKH, 128), lambda i: (i, 0)),
        pl.BlockSpec((2, 128), lambda i: (0, 0)),
        pl.BlockSpec((BLKH, 4), lambda i: (i, 0)),
        pl.BlockSpec((HID, N_CLASSES), lambda i: (0, 0)),
        pl.BlockSpec((1, N_CLASSES), lambda i: (0, 0)),
    ],
    out_specs=pl.BlockSpec((N_GRAPHS, N_CLASSES), lambda i: (0, 0)),
    out_shape=jax.ShapeDtypeStruct((N_GRAPHS, N_CLASSES), jnp.float32),
    scratch_shapes=[pltpu.VMEM((N_GRAPHS, 264), jnp.float32)],
)


def _bd(W):
  eye4 = jnp.eye(4, dtype=W.dtype)
  return jnp.stack([
      jnp.stack([jnp.kron(eye4, W[i * 32:(i + 1) * 32, o * 32:(o + 1) * 32])
                 for o in range(2)])
      for i in range(2)])


def _bt(b):
  return jnp.stack([jnp.tile(b[:32], 4), jnp.tile(b[32:], 4)])


@jax.jit
def kernel(x, edge_index, batch, W1, b1, W2, b2, W3, b3, Wc, bc):
  row = edge_index[0].astype(jnp.int32)
  col = edge_index[1].astype(jnp.int32)
  npad = E_PAD - N_EDGES
  junkc = N_NODES + (jnp.arange(npad, dtype=jnp.int32) % (NPAD - N_NODES))
  row2d = jnp.concatenate([row, jnp.zeros((npad,), jnp.int32)]).reshape(
      N_CHUNKS, CHUNK)
  col2d = jnp.concatenate([col, junkc]).reshape(N_CHUNKS, CHUNK)
  bjs = jnp.concatenate(
      [batch.astype(jnp.int32),
       jnp.full((NPAD - N_NODES,), -1, jnp.int32)]).reshape(HROWS, 4)

  e0 = jnp.zeros((CHUNK, 16), jnp.float32).at[:, 0].set(1.0)
  zerosD = jnp.zeros((NPAD, 16), jnp.float32)

  def pack(a):
    return a.reshape(NC, HROWS, 128)

  def unpack(a):
    return a.reshape(NC, NPAD, HID // 2)

  deg = _deg_call(col2d, e0, zerosD)
  h1 = _mm1(x, W1)
  hs1, dinv32 = _scale1(h1, deg)
  dinvP = dinv32.reshape(HROWS, 128)
  acc1 = _agg_call(hs1, row2d, col2d)
  hs2 = _layer(pack(acc1), dinvP, _bd(W2), _bt(b1))
  acc2 = _agg_call(unpack(hs2), row2d, col2d)
  hs3 = _layer(pack(acc2), dinvP, _bd(W3), _bt(b2))
  acc3 = _agg_call(unpack(hs3), row2d, col2d)
  return _pool(pack(acc3), dinvP, _bt(b3), bjs,
               Wc, bc.reshape(1, N_CLASSES))


# speedup vs baseline: 1.1982x; 1.0548x over previous
import jax
import jax.numpy as jnp
from jax import lax
from jax.experimental import pallas as pl
from jax.experimental.pallas import tpu as pltpu
from jax.experimental.pallas import tpu_sc as plsc

N_NODES = 50000
N_EDGES = 800000
IN_DIM = 128
HID = 64
N_CLASSES = 8
N_GRAPHS = 64

NC = 2
NS = 16
CHUNK = 128
GROUP = 4
GROUPS = 98
DEG_GROUP = 4
N_CHUNKS = NS * GROUP * GROUPS
E_PAD = N_CHUNKS * CHUNK
NPAD = 50176
HROWS = NPAD * 32 // 128
DROWS = NPAD * 16 // 128
BROWS = NPAD // 128
SUB_H = HROWS // NS
SUB_D = DROWS // NS
DEG_GROUPS = N_CHUNKS // (NC * NS * DEG_GROUP)

GRID = 7
BLKN = NPAD // GRID
BLKH = HROWS // GRID
BLKD = DROWS // GRID
BLKB = BROWS // GRID

_mesh = plsc.VectorSubcoreMesh(core_axis_name="c", subcore_axis_name="s")
_sc_params = pltpu.CompilerParams(use_tc_tiling_on_sc=False)

_HIGH = jax.lax.Precision.HIGHEST


def _dot(a, b):
  return jax.lax.dot_general(a, b, (((1,), (0,)), ((), ())),
                             precision=_HIGH,
                             preferred_element_type=jnp.float32)


def _deg_body(col_hbm, e0_hbm, zeros_hbm, deg_hbm, cbuf, valbuf, acc, sem):
  c = lax.axis_index("c")
  s = lax.axis_index("s")
  nb = pl.multiple_of(s * (NPAD // NS), 8)
  pltpu.sync_copy(zeros_hbm.at[pl.ds(nb, NPAD // NS)],
                  acc.at[pl.ds(nb, NPAD // NS)])
  pltpu.sync_copy(e0_hbm, valbuf)
  plsc.subcore_barrier()

  base0 = c * (N_CHUNKS // NC) + s * (DEG_GROUPS * DEG_GROUP)

  @pl.loop(0, DEG_GROUPS)
  def _(g):
    base = base0 + g * DEG_GROUP
    pltpu.sync_copy(col_hbm.at[pl.ds(base, DEG_GROUP)], cbuf)
    waits = []
    for j in range(DEG_GROUP):
      waits.append(pltpu.async_copy(valbuf, acc.at[cbuf.at[j]], sem, add=True))
    for w in waits:
      w.wait()

  plsc.subcore_barrier()
  pltpu.sync_copy(acc.at[pl.ds(nb, NPAD // NS)],
                  deg_hbm.at[c, pl.ds(nb, NPAD // NS)])


_deg_call = pl.kernel(
    _deg_body,
    out_type=jax.ShapeDtypeStruct((NC, NPAD, 16), jnp.float32),
    mesh=_mesh,
    scratch_types=[
        pltpu.VMEM((DEG_GROUP, CHUNK), jnp.int32),
        pltpu.VMEM((CHUNK, 16), jnp.float32),
        pltpu.VMEM_SHARED((NPAD, 16), jnp.float32),
        pltpu.SemaphoreType.DMA,
    ],
    compiler_params=_sc_params,
)


def _agg_body(hs_hbm, row_hbm, col_hbm, out_hbm,
              rbuf, cbuf, v0, v1, v2, v3, acc, sem_g, sem_s):
  vals = [v0, v1, v2, v3]
  c = lax.axis_index("c")
  s = lax.axis_index("s")
  nb = pl.multiple_of(s * (NPAD // NS), 8)
  pltpu.sync_copy(hs_hbm.at[c, pl.ds(nb, NPAD // NS)],
                  acc.at[pl.ds(nb, NPAD // NS)])
  plsc.subcore_barrier()

  base0 = s * (GROUP * GROUPS)

  def do_batches(base, nhalf):
    pltpu.sync_copy(row_hbm.at[pl.ds(base, nhalf * GROUP)],
                    rbuf.at[pl.ds(0, nhalf * GROUP)])
    pltpu.sync_copy(col_hbm.at[pl.ds(base, nhalf * GROUP)],
                    cbuf.at[pl.ds(0, nhalf * GROUP)])
    for half in range(nhalf):
      gathers = []
      for k in range(GROUP):
        gathers.append(pltpu.async_copy(
            hs_hbm.at[c].at[rbuf.at[half * GROUP + k]], vals[k], sem_g))
      for w in gathers:
        w.wait()
      scatters = []
      for k in range(GROUP):
        scatters.append(pltpu.async_copy(
            vals[k], acc.at[cbuf.at[half * GROUP + k]], sem_s, add=True))
      for w in scatters:
        w.wait()

  @pl.loop(0, 24)
  def _(g):
    do_batches(base0 + g * (4 * GROUP), 4)

  do_batches(base0 + 24 * 4 * GROUP, 2)

  plsc.subcore_barrier()
  pltpu.sync_copy(acc.at[pl.ds(nb, NPAD // NS)],
                  out_hbm.at[c, pl.ds(nb, NPAD // NS)])


_agg_call = pl.kernel(
    _agg_body,
    out_type=jax.ShapeDtypeStruct((NC, NPAD, HID // 2), jnp.float32),
    mesh=_mesh,
    scratch_types=(
        [pltpu.VMEM((4 * GROUP, CHUNK), jnp.int32)] * 2
        + [pltpu.VMEM((CHUNK, HID // 2), jnp.float32)] * GROUP
        + [pltpu.VMEM_SHARED((NPAD, HID // 2), jnp.float32)]
        + [pltpu.SemaphoreType.DMA] * 2
    ),
    compiler_params=_sc_params,
)


def _mm1_body(x_ref, w_ref, o_ref):
  o_ref[...] = _dot(x_ref[...], w_ref[...])


_mm1 = pl.pallas_call(
    _mm1_body,
    grid=(25,),
    in_specs=[
        pl.BlockSpec((2000, IN_DIM), lambda i: (i, 0)),
        pl.BlockSpec((IN_DIM, HID), lambda i: (0, 0)),
    ],
    out_specs=pl.BlockSpec((2000, HID), lambda i: (i, 0)),
    out_shape=jax.ShapeDtypeStruct((N_NODES, HID), jnp.float32),
)


def _scale1_body(h_ref, deg_ref, hs_ref, dinv_ref):
  deg = deg_ref[0] + deg_ref[1]
  tot = 1.0 + jnp.sum(deg, axis=1)
  dinv = jax.lax.rsqrt(tot)[:, None]
  i = pl.program_id(0)
  nid = i * BLKN + jax.lax.broadcasted_iota(jnp.int32, (BLKN, 1), 0)
  hs = jnp.where(nid < N_NODES, h_ref[...] * dinv, 0.0)
  hs_ref[0] = hs[:, :HID // 2]
  hs_ref[1] = hs[:, HID // 2:]
  dinv_ref[...] = jnp.broadcast_to(dinv, (BLKN, 32))


_scale1 = pl.pallas_call(
    _scale1_body,
    grid=(GRID,),
    in_specs=[
        pl.BlockSpec((BLKN, HID), lambda i: (i, 0)),
        pl.BlockSpec((NC, BLKN, 16), lambda i: (0, i, 0)),
    ],
    out_specs=[
        pl.BlockSpec((NC, BLKN, HID // 2), lambda i: (0, i, 0)),
        pl.BlockSpec((BLKN, 32), lambda i: (i, 0)),
    ],
    out_shape=[
        jax.ShapeDtypeStruct((NC, NPAD, HID // 2), jnp.float32),
        jax.ShapeDtypeStruct((NPAD, 32), jnp.float32),
    ],
)


def _layer_body(acc_ref, dinv_ref, w_ref, b_ref, hs_ref):
  d = dinv_ref[...]
  r0 = jnp.maximum(acc_ref[0] * d + b_ref[0:1, :], 0.0)
  r1 = jnp.maximum(acc_ref[1] * d + b_ref[1:2, :], 0.0)
  h0 = _dot(r0, w_ref[0, 0]) + _dot(r1, w_ref[1, 0])
  h1 = _dot(r0, w_ref[0, 1]) + _dot(r1, w_ref[1, 1])
  hs_ref[0] = h0 * d
  hs_ref[1] = h1 * d


_layer = pl.pallas_call(
    _layer_body,
    grid=(GRID,),
    in_specs=[
        pl.BlockSpec((NC, BLKH, 128), lambda i: (0, i, 0)),
        pl.BlockSpec((BLKH, 128), lambda i: (i, 0)),
        pl.BlockSpec((2, 2, 128, 128), lambda i: (0, 0, 0, 0)),
        pl.BlockSpec((2, 128), lambda i: (0, 0)),
    ],
    out_specs=pl.BlockSpec((NC, BLKH, 128), lambda i: (0, i, 0)),
    out_shape=jax.ShapeDtypeStruct((NC, HROWS, 128), jnp.float32),
)


def _pool_body(acc_ref, dinv_ref, b_ref, bjs_ref, wc_ref, bc_ref,
               out_ref, pool_scr):
  i = pl.program_id(0)

  @pl.when(i == 0)
  def _():
    pool_scr[...] = jnp.zeros_like(pool_scr)

  d = dinv_ref[...]
  r0 = jnp.maximum(acc_ref[0] * d + b_ref[0:1, :], 0.0)
  r1 = jnp.maximum(acc_ref[1] * d + b_ref[1:2, :], 0.0)
  laneblk = jax.lax.broadcasted_iota(jnp.int32, (BLKH, 128), 1) // 32
  giota = jax.lax.broadcasted_iota(jnp.int32, (BLKH, N_GRAPHS), 1)
  for j in range(4):
    mj = laneblk == j
    zc = jnp.concatenate(
        [jnp.where(mj, r0, 0.0), jnp.where(mj, r1, 0.0),
         jnp.ones((BLKH, 8), jnp.float32)], axis=1)
    ohj = (bjs_ref[:, j:j + 1] == giota).astype(jnp.float32)
    pool_scr[...] += jax.lax.dot_general(
        ohj, zc, (((0,), (0,)), ((), ())),
        precision=_HIGH, preferred_element_type=jnp.float32)

  @pl.when(i == GRID - 1)
  def _():
    pool = pool_scr[...]
    l32 = jax.lax.broadcasted_iota(jnp.int32, (128, 32), 0) % 32
    fold = (l32 == jax.lax.broadcasted_iota(jnp.int32, (128, 32), 1)
            ).astype(jnp.float32)
    sums = jnp.concatenate(
        [_dot(pool[:, :128], fold), _dot(pool[:, 128:256], fold)], axis=1)
    counts = jnp.maximum(pool[:, 256:257], 1.0)
    out_ref[...] = _dot(sums / counts, wc_ref[...]) + bc_ref[...]


_pool = pl.pallas_call(
    _pool_body,
    grid=(GRID,),
    in_specs=[
        pl.BlockSpec((NC, BLKH, 128), lambda i: (0, i, 0)),
        pl.BlockSpec((BLKH, 128), lambda i: (i, 0)),
        pl.BlockSpec((2, 128), lambda i: (0, 0)),
        pl.BlockSpec((BLKH, 4), lambda i: (i, 0)),
        pl.BlockSpec((HID, N_CLASSES), lambda i: (0, 0)),
        pl.BlockSpec((1, N_CLASSES), lambda i: (0, 0)),
    ],
    out_specs=pl.BlockSpec((N_GRAPHS, N_CLASSES), lambda i: (0, 0)),
    out_shape=jax.ShapeDtypeStruct((N_GRAPHS, N_CLASSES), jnp.float32),
    scratch_shapes=[pltpu.VMEM((N_GRAPHS, 264), jnp.float32)],
)


def _bd(W):
  eye4 = jnp.eye(4, dtype=W.dtype)
  return jnp.stack([
      jnp.stack([jnp.kron(eye4, W[i * 32:(i + 1) * 32, o * 32:(o + 1) * 32])
                 for o in range(2)])
      for i in range(2)])


def _bt(b):
  return jnp.stack([jnp.tile(b[:32], 4), jnp.tile(b[32:], 4)])


@jax.jit
def kernel(x, edge_index, batch, W1, b1, W2, b2, W3, b3, Wc, bc):
  row = edge_index[0].astype(jnp.int32)
  col = edge_index[1].astype(jnp.int32)
  npad = E_PAD - N_EDGES
  junkc = N_NODES + (jnp.arange(npad, dtype=jnp.int32) % (NPAD - N_NODES))
  row2d = jnp.concatenate([row, jnp.zeros((npad,), jnp.int32)]).reshape(
      N_CHUNKS, CHUNK)
  col2d = jnp.concatenate([col, junkc]).reshape(N_CHUNKS, CHUNK)
  bjs = jnp.concatenate(
      [batch.astype(jnp.int32),
       jnp.full((NPAD - N_NODES,), -1, jnp.int32)]).reshape(HROWS, 4)

  e0 = jnp.zeros((CHUNK, 16), jnp.float32).at[:, 0].set(1.0)
  zerosD = jnp.zeros((NPAD, 16), jnp.float32)

  def pack(a):
    return a.reshape(NC, HROWS, 128)

  def unpack(a):
    return a.reshape(NC, NPAD, HID // 2)

  deg = _deg_call(col2d, e0, zerosD)
  h1 = _mm1(x, W1)
  hs1, dinv32 = _scale1(h1, deg)
  dinvP = dinv32.reshape(HROWS, 128)
  acc1 = _agg_call(hs1, row2d, col2d)
  hs2 = _layer(pack(acc1), dinvP, _bd(W2), _bt(b1))
  acc2 = _agg_call(unpack(hs2), row2d, col2d)
  hs3 = _layer(pack(acc2), dinvP, _bd(W3), _bt(b2))
  acc3 = _agg_call(unpack(hs3), row2d, col2d)
  return _pool(pack(acc3), dinvP, _bt(b3), bjs,
               Wc, bc.reshape(1, N_CLASSES))


# speedup vs baseline: 1.2192x; 1.0175x over previous
import jax
import jax.numpy as jnp
from jax import lax
from jax.experimental import pallas as pl
from jax.experimental.pallas import tpu as pltpu
from jax.experimental.pallas import tpu_sc as plsc

N_NODES = 50000
N_EDGES = 800000
IN_DIM = 128
HID = 64
N_CLASSES = 8
N_GRAPHS = 64

NC = 2
NS = 16
CHUNK = 128
GROUP = 4
GROUPS = 98
DEG_GROUP = 4
N_CHUNKS = NS * GROUP * GROUPS
E_PAD = N_CHUNKS * CHUNK
NPAD = 50176
HROWS = NPAD * 32 // 128
DROWS = NPAD * 16 // 128
BROWS = NPAD // 128
SUB_H = HROWS // NS
SUB_D = DROWS // NS
DEG_GROUPS = N_CHUNKS // (NC * NS * DEG_GROUP)

GRID = 7
BLKN = NPAD // GRID
BLKH = HROWS // GRID
BLKD = DROWS // GRID
BLKB = BROWS // GRID

_mesh = plsc.VectorSubcoreMesh(core_axis_name="c", subcore_axis_name="s")
_sc_params = pltpu.CompilerParams(use_tc_tiling_on_sc=False)

_HIGH = jax.lax.Precision.HIGHEST


def _dot(a, b):
  return jax.lax.dot_general(a, b, (((1,), (0,)), ((), ())),
                             precision=_HIGH,
                             preferred_element_type=jnp.float32)


def _deg_body(col_hbm, e0_hbm, zeros_hbm, deg_hbm, cbuf, valbuf, acc, sem):
  c = lax.axis_index("c")
  s = lax.axis_index("s")
  nb = pl.multiple_of(s * (NPAD // NS), 8)
  pltpu.sync_copy(zeros_hbm.at[pl.ds(nb, NPAD // NS)],
                  acc.at[pl.ds(nb, NPAD // NS)])
  pltpu.sync_copy(e0_hbm, valbuf)
  plsc.subcore_barrier()

  base0 = c * (N_CHUNKS // NC) + s * (DEG_GROUPS * DEG_GROUP)

  def deg_batches(base, nhalf):
    pltpu.sync_copy(col_hbm.at[pl.ds(base, nhalf * DEG_GROUP)],
                    cbuf.at[pl.ds(0, nhalf * DEG_GROUP)])
    for half in range(nhalf):
      waits = []
      for j in range(DEG_GROUP):
        waits.append(pltpu.async_copy(
            valbuf, acc.at[cbuf.at[half * DEG_GROUP + j]], sem, add=True))
      for w in waits:
        w.wait()

  @pl.loop(0, 12)
  def _(g):
    deg_batches(base0 + g * (4 * DEG_GROUP), 4)

  deg_batches(base0 + 12 * 4 * DEG_GROUP, 1)

  plsc.subcore_barrier()
  pltpu.sync_copy(acc.at[pl.ds(nb, NPAD // NS)],
                  deg_hbm.at[c, pl.ds(nb, NPAD // NS)])


_deg_call = pl.kernel(
    _deg_body,
    out_type=jax.ShapeDtypeStruct((NC, NPAD, 16), jnp.float32),
    mesh=_mesh,
    scratch_types=[
        pltpu.VMEM((4 * DEG_GROUP, CHUNK), jnp.int32),
        pltpu.VMEM((CHUNK, 16), jnp.float32),
        pltpu.VMEM_SHARED((NPAD, 16), jnp.float32),
        pltpu.SemaphoreType.DMA,
    ],
    compiler_params=_sc_params,
)


def _agg_body(hs_hbm, row_hbm, col_hbm, out_hbm,
              rbuf, cbuf, v0, v1, v2, v3, acc, sem_g, sem_s):
  vals = [v0, v1, v2, v3]
  c = lax.axis_index("c")
  s = lax.axis_index("s")
  nb = pl.multiple_of(s * (NPAD // NS), 8)
  pltpu.sync_copy(hs_hbm.at[c, pl.ds(nb, NPAD // NS)],
                  acc.at[pl.ds(nb, NPAD // NS)])
  plsc.subcore_barrier()

  base0 = s * (GROUP * GROUPS)

  def do_batches(base, nhalf):
    pltpu.sync_copy(row_hbm.at[pl.ds(base, nhalf * GROUP)],
                    rbuf.at[pl.ds(0, nhalf * GROUP)])
    pltpu.sync_copy(col_hbm.at[pl.ds(base, nhalf * GROUP)],
                    cbuf.at[pl.ds(0, nhalf * GROUP)])
    for half in range(nhalf):
      gathers = []
      for k in range(GROUP):
        gathers.append(pltpu.async_copy(
            hs_hbm.at[c].at[rbuf.at[half * GROUP + k]], vals[k], sem_g))
      for w in gathers:
        w.wait()
      scatters = []
      for k in range(GROUP):
        scatters.append(pltpu.async_copy(
            vals[k], acc.at[cbuf.at[half * GROUP + k]], sem_s, add=True))
      for w in scatters:
        w.wait()

  @pl.loop(0, 24)
  def _(g):
    do_batches(base0 + g * (4 * GROUP), 4)

  do_batches(base0 + 24 * 4 * GROUP, 2)

  plsc.subcore_barrier()
  pltpu.sync_copy(acc.at[pl.ds(nb, NPAD // NS)],
                  out_hbm.at[c, pl.ds(nb, NPAD // NS)])


_agg_call = pl.kernel(
    _agg_body,
    out_type=jax.ShapeDtypeStruct((NC, NPAD, HID // 2), jnp.float32),
    mesh=_mesh,
    scratch_types=(
        [pltpu.VMEM((4 * GROUP, CHUNK), jnp.int32)] * 2
        + [pltpu.VMEM((CHUNK, HID // 2), jnp.float32)] * GROUP
        + [pltpu.VMEM_SHARED((NPAD, HID // 2), jnp.float32)]
        + [pltpu.SemaphoreType.DMA] * 2
    ),
    compiler_params=_sc_params,
)


def _mm1_body(x_ref, w_ref, o_ref):
  o_ref[...] = _dot(x_ref[...], w_ref[...])


_mm1 = pl.pallas_call(
    _mm1_body,
    grid=(25,),
    in_specs=[
        pl.BlockSpec((2000, IN_DIM), lambda i: (i, 0)),
        pl.BlockSpec((IN_DIM, HID), lambda i: (0, 0)),
    ],
    out_specs=pl.BlockSpec((2000, HID), lambda i: (i, 0)),
    out_shape=jax.ShapeDtypeStruct((N_NODES, HID), jnp.float32),
)


def _scale1_body(h_ref, deg_ref, hs_ref, dinv_ref):
  deg = deg_ref[0] + deg_ref[1]
  tot = 1.0 + jnp.sum(deg, axis=1)
  dinv = jax.lax.rsqrt(tot)[:, None]
  i = pl.program_id(0)
  nid = i * BLKN + jax.lax.broadcasted_iota(jnp.int32, (BLKN, 1), 0)
  hs = jnp.where(nid < N_NODES, h_ref[...] * dinv, 0.0)
  hs_ref[0] = hs[:, :HID // 2]
  hs_ref[1] = hs[:, HID // 2:]
  dinv_ref[...] = jnp.broadcast_to(dinv, (BLKN, 32))


_scale1 = pl.pallas_call(
    _scale1_body,
    grid=(GRID,),
    in_specs=[
        pl.BlockSpec((BLKN, HID), lambda i: (i, 0)),
        pl.BlockSpec((NC, BLKN, 16), lambda i: (0, i, 0)),
    ],
    out_specs=[
        pl.BlockSpec((NC, BLKN, HID // 2), lambda i: (0, i, 0)),
        pl.BlockSpec((BLKN, 32), lambda i: (i, 0)),
    ],
    out_shape=[
        jax.ShapeDtypeStruct((NC, NPAD, HID // 2), jnp.float32),
        jax.ShapeDtypeStruct((NPAD, 32), jnp.float32),
    ],
)


def _layer_body(acc_ref, dinv_ref, w_ref, b_ref, hs_ref):
  d = dinv_ref[...]
  r0 = jnp.maximum(acc_ref[0] * d + b_ref[0:1, :], 0.0)
  r1 = jnp.maximum(acc_ref[1] * d + b_ref[1:2, :], 0.0)
  h0 = _dot(r0, w_ref[0, 0]) + _dot(r1, w_ref[1, 0])
  h1 = _dot(r0, w_ref[0, 1]) + _dot(r1, w_ref[1, 1])
  hs_ref[0] = h0 * d
  hs_ref[1] = h1 * d


_layer = pl.pallas_call(
    _layer_body,
    grid=(GRID,),
    in_specs=[
        pl.BlockSpec((NC, BLKH, 128), lambda i: (0, i, 0)),
        pl.BlockSpec((BLKH, 128), lambda i: (i, 0)),
        pl.BlockSpec((2, 2, 128, 128), lambda i: (0, 0, 0, 0)),
        pl.BlockSpec((2, 128), lambda i: (0, 0)),
    ],
    out_specs=pl.BlockSpec((NC, BLKH, 128), lambda i: (0, i, 0)),
    out_shape=jax.ShapeDtypeStruct((NC, HROWS, 128), jnp.float32),
)


def _pool_body(acc_ref, dinv_ref, b_ref, bjs_ref, wc_ref, bc_ref,
               out_ref, pool_scr):
  i = pl.program_id(0)

  @pl.when(i == 0)
  def _():
    pool_scr[...] = jnp.zeros_like(pool_scr)

  d = dinv_ref[...]
  r0 = jnp.maximum(acc_ref[0] * d + b_ref[0:1, :], 0.0)
  r1 = jnp.maximum(acc_ref[1] * d + b_ref[1:2, :], 0.0)
  laneblk = jax.lax.broadcasted_iota(jnp.int32, (BLKH, 128), 1) // 32
  giota = jax.lax.broadcasted_iota(jnp.int32, (BLKH, N_GRAPHS), 1)
  for j in range(4):
    mj = laneblk == j
    zc = jnp.concatenate(
        [jnp.where(mj, r0, 0.0), jnp.where(mj, r1, 0.0),
         jnp.ones((BLKH, 8), jnp.float32)], axis=1)
    ohj = (bjs_ref[:, j:j + 1] == giota).astype(jnp.float32)
    pool_scr[...] += jax.lax.dot_general(
        ohj, zc, (((0,), (0,)), ((), ())),
        precision=_HIGH, preferred_element_type=jnp.float32)

  @pl.when(i == GRID - 1)
  def _():
    pool = pool_scr[...]
    l32 = jax.lax.broadcasted_iota(jnp.int32, (128, 32), 0) % 32
    fold = (l32 == jax.lax.broadcasted_iota(jnp.int32, (128, 32), 1)
            ).astype(jnp.float32)
    sums = jnp.concatenate(
        [_dot(pool[:, :128], fold), _dot(pool[:, 128:256], fold)], axis=1)
    counts = jnp.maximum(pool[:, 256:257], 1.0)
    out_ref[...] = _dot(sums / counts, wc_ref[...]) + bc_ref[...]


_pool = pl.pallas_call(
    _pool_body,
    grid=(GRID,),
    in_specs=[
        pl.BlockSpec((NC, BLKH, 128), lambda i: (0, i, 0)),
        pl.BlockSpec((BLKH, 128), lambda i: (i, 0)),
        pl.BlockSpec((2, 128), lambda i: (0, 0)),
        pl.BlockSpec((BLKH, 4), lambda i: (i, 0)),
        pl.BlockSpec((HID, N_CLASSES), lambda i: (0, 0)),
        pl.BlockSpec((1, N_CLASSES), lambda i: (0, 0)),
    ],
    out_specs=pl.BlockSpec((N_GRAPHS, N_CLASSES), lambda i: (0, 0)),
    out_shape=jax.ShapeDtypeStruct((N_GRAPHS, N_CLASSES), jnp.float32),
    scratch_shapes=[pltpu.VMEM((N_GRAPHS, 264), jnp.float32)],
)


def _bd(W):
  eye4 = jnp.eye(4, dtype=W.dtype)
  return jnp.stack([
      jnp.stack([jnp.kron(eye4, W[i * 32:(i + 1) * 32, o * 32:(o + 1) * 32])
                 for o in range(2)])
      for i in range(2)])


def _bt(b):
  return jnp.stack([jnp.tile(b[:32], 4), jnp.tile(b[32:], 4)])


@jax.jit
def kernel(x, edge_index, batch, W1, b1, W2, b2, W3, b3, Wc, bc):
  row = edge_index[0].astype(jnp.int32)
  col = edge_index[1].astype(jnp.int32)
  npad = E_PAD - N_EDGES
  junkc = N_NODES + (jnp.arange(npad, dtype=jnp.int32) % (NPAD - N_NODES))
  row2d = jnp.concatenate([row, jnp.zeros((npad,), jnp.int32)]).reshape(
      N_CHUNKS, CHUNK)
  col2d = jnp.concatenate([col, junkc]).reshape(N_CHUNKS, CHUNK)
  bjs = jnp.concatenate(
      [batch.astype(jnp.int32),
       jnp.full((NPAD - N_NODES,), -1, jnp.int32)]).reshape(HROWS, 4)

  e0 = jnp.zeros((CHUNK, 16), jnp.float32).at[:, 0].set(1.0)
  zerosD = jnp.zeros((NPAD, 16), jnp.float32)

  def pack(a):
    return a.reshape(NC, HROWS, 128)

  def unpack(a):
    return a.reshape(NC, NPAD, HID // 2)

  deg = _deg_call(col2d, e0, zerosD)
  h1 = _mm1(x, W1)
  hs1, dinv32 = _scale1(h1, deg)
  dinvP = dinv32.reshape(HROWS, 128)
  acc1 = _agg_call(hs1, row2d, col2d)
  hs2 = _layer(pack(acc1), dinvP, _bd(W2), _bt(b1))
  acc2 = _agg_call(unpack(hs2), row2d, col2d)
  hs3 = _layer(pack(acc2), dinvP, _bd(W3), _bt(b2))
  acc3 = _agg_call(unpack(hs3), row2d, col2d)
  return _pool(pack(acc3), dinvP, _bt(b3), bjs,
               Wc, bc.reshape(1, N_CLASSES))


# speedup vs baseline: 1.2568x; 1.0309x over previous
import jax
import jax.numpy as jnp
from jax import lax
from jax.experimental import pallas as pl
from jax.experimental.pallas import tpu as pltpu
from jax.experimental.pallas import tpu_sc as plsc

N_NODES = 50000
N_EDGES = 800000
IN_DIM = 128
HID = 64
N_CLASSES = 8
N_GRAPHS = 64

NC = 2
NS = 16
CHUNK = 128
GROUP = 4
GROUPS = 98
DEG_GROUP = 4
N_CHUNKS = NS * GROUP * GROUPS
E_PAD = N_CHUNKS * CHUNK
NPAD = 50176
HROWS = NPAD * 32 // 128
DROWS = NPAD * 16 // 128
BROWS = NPAD // 128
SUB_H = HROWS // NS
SUB_D = DROWS // NS
DEG_GROUPS = N_CHUNKS // (NC * NS * DEG_GROUP)

GRID = 7
BLKN = NPAD // GRID
BLKH = HROWS // GRID
BLKD = DROWS // GRID
BLKB = BROWS // GRID

_mesh = plsc.VectorSubcoreMesh(core_axis_name="c", subcore_axis_name="s")
_sc_params = pltpu.CompilerParams(use_tc_tiling_on_sc=False)

_HIGH = jax.lax.Precision.HIGHEST


def _dot(a, b):
  return jax.lax.dot_general(a, b, (((1,), (0,)), ((), ())),
                             precision=_HIGH,
                             preferred_element_type=jnp.float32)


def _deg_body(col_hbm, e0_hbm, zeros_hbm, deg_hbm, cbuf, valbuf, acc, sem):
  c = lax.axis_index("c")
  s = lax.axis_index("s")
  nb = pl.multiple_of(s * (NPAD // NS), 8)
  pltpu.sync_copy(zeros_hbm.at[pl.ds(nb, NPAD // NS)],
                  acc.at[pl.ds(nb, NPAD // NS)])
  pltpu.sync_copy(e0_hbm, valbuf)
  plsc.subcore_barrier()

  base0 = c * (N_CHUNKS // NC) + s * (DEG_GROUPS * DEG_GROUP)

  def deg_batches(base, nhalf):
    pltpu.sync_copy(col_hbm.at[pl.ds(base, nhalf * DEG_GROUP)],
                    cbuf.at[pl.ds(0, nhalf * DEG_GROUP)])
    for half in range(nhalf):
      waits = []
      for j in range(DEG_GROUP):
        waits.append(pltpu.async_copy(
            valbuf, acc.at[cbuf.at[half * DEG_GROUP + j]], sem, add=True))
      for w in waits:
        w.wait()

  @pl.loop(0, 12)
  def _(g):
    deg_batches(base0 + g * (4 * DEG_GROUP), 4)

  deg_batches(base0 + 12 * 4 * DEG_GROUP, 1)

  plsc.subcore_barrier()
  pltpu.sync_copy(acc.at[pl.ds(nb, NPAD // NS)],
                  deg_hbm.at[c, pl.ds(nb, NPAD // NS)])


_deg_call = pl.kernel(
    _deg_body,
    out_type=jax.ShapeDtypeStruct((NC, NPAD, 16), jnp.float32),
    mesh=_mesh,
    scratch_types=[
        pltpu.VMEM((4 * DEG_GROUP, CHUNK), jnp.int32),
        pltpu.VMEM((CHUNK, 16), jnp.float32),
        pltpu.VMEM_SHARED((NPAD, 16), jnp.float32),
        pltpu.SemaphoreType.DMA,
    ],
    compiler_params=_sc_params,
)


def _agg_body(hs_hbm, row_hbm, col_hbm, out_hbm,
              rbuf, cbuf, v0, v1, v2, v3, acc, sem_g, sem_s):
  vals = [v0, v1, v2, v3]
  c = lax.axis_index("c")
  s = lax.axis_index("s")
  nb = pl.multiple_of(s * (NPAD // NS), 8)
  pltpu.sync_copy(hs_hbm.at[c, pl.ds(nb, NPAD // NS)],
                  acc.at[pl.ds(nb, NPAD // NS)])
  plsc.subcore_barrier()

  base0 = s * (GROUP * GROUPS)

  def do_batches(base, nhalf):
    pltpu.sync_copy(row_hbm.at[pl.ds(base, nhalf * GROUP)],
                    rbuf.at[pl.ds(0, nhalf * GROUP)])
    pltpu.sync_copy(col_hbm.at[pl.ds(base, nhalf * GROUP)],
                    cbuf.at[pl.ds(0, nhalf * GROUP)])
    for half in range(nhalf):
      gathers = []
      for k in range(GROUP):
        gathers.append(pltpu.async_copy(
            hs_hbm.at[c].at[rbuf.at[half * GROUP + k]], vals[k], sem_g))
      for w in gathers:
        w.wait()
      scatters = []
      for k in range(GROUP):
        scatters.append(pltpu.async_copy(
            vals[k], acc.at[cbuf.at[half * GROUP + k]], sem_s, add=True))
      for w in scatters:
        w.wait()

  @pl.loop(0, 12)
  def _(g):
    do_batches(base0 + g * (8 * GROUP), 8)

  do_batches(base0 + 12 * 8 * GROUP, 2)

  plsc.subcore_barrier()
  pltpu.sync_copy(acc.at[pl.ds(nb, NPAD // NS)],
                  out_hbm.at[c, pl.ds(nb, NPAD // NS)])


_agg_call = pl.kernel(
    _agg_body,
    out_type=jax.ShapeDtypeStruct((NC, NPAD, HID // 2), jnp.float32),
    mesh=_mesh,
    scratch_types=(
        [pltpu.VMEM((8 * GROUP, CHUNK), jnp.int32)] * 2
        + [pltpu.VMEM((CHUNK, HID // 2), jnp.float32)] * GROUP
        + [pltpu.VMEM_SHARED((NPAD, HID // 2), jnp.float32)]
        + [pltpu.SemaphoreType.DMA] * 2
    ),
    compiler_params=_sc_params,
)


def _mm1_body(x_ref, w_ref, o_ref):
  o_ref[...] = _dot(x_ref[...], w_ref[...])


_mm1 = pl.pallas_call(
    _mm1_body,
    grid=(25,),
    in_specs=[
        pl.BlockSpec((2000, IN_DIM), lambda i: (i, 0)),
        pl.BlockSpec((IN_DIM, HID), lambda i: (0, 0)),
    ],
    out_specs=pl.BlockSpec((2000, HID), lambda i: (i, 0)),
    out_shape=jax.ShapeDtypeStruct((N_NODES, HID), jnp.float32),
)


def _scale1_body(h_ref, deg_ref, hs_ref, dinv_ref):
  deg = deg_ref[0] + deg_ref[1]
  tot = 1.0 + jnp.sum(deg, axis=1)
  dinv = jax.lax.rsqrt(tot)[:, None]
  i = pl.program_id(0)
  nid = i * BLKN + jax.lax.broadcasted_iota(jnp.int32, (BLKN, 1), 0)
  hs = jnp.where(nid < N_NODES, h_ref[...] * dinv, 0.0)
  hs_ref[0] = hs[:, :HID // 2]
  hs_ref[1] = hs[:, HID // 2:]
  dinv_ref[...] = jnp.broadcast_to(dinv, (BLKN, 32))


_scale1 = pl.pallas_call(
    _scale1_body,
    grid=(GRID,),
    in_specs=[
        pl.BlockSpec((BLKN, HID), lambda i: (i, 0)),
        pl.BlockSpec((NC, BLKN, 16), lambda i: (0, i, 0)),
    ],
    out_specs=[
        pl.BlockSpec((NC, BLKN, HID // 2), lambda i: (0, i, 0)),
        pl.BlockSpec((BLKN, 32), lambda i: (i, 0)),
    ],
    out_shape=[
        jax.ShapeDtypeStruct((NC, NPAD, HID // 2), jnp.float32),
        jax.ShapeDtypeStruct((NPAD, 32), jnp.float32),
    ],
)


def _layer_body(acc_ref, dinv_ref, w_ref, b_ref, hs_ref):
  d = dinv_ref[...]
  r0 = jnp.maximum(acc_ref[0] * d + b_ref[0:1, :], 0.0)
  r1 = jnp.maximum(acc_ref[1] * d + b_ref[1:2, :], 0.0)
  h0 = _dot(r0, w_ref[0, 0]) + _dot(r1, w_ref[1, 0])
  h1 = _dot(r0, w_ref[0, 1]) + _dot(r1, w_ref[1, 1])
  hs_ref[0] = h0 * d
  hs_ref[1] = h1 * d


_layer = pl.pallas_call(
    _layer_body,
    grid=(GRID,),
    in_specs=[
        pl.BlockSpec((NC, BLKH, 128), lambda i: (0, i, 0)),
        pl.BlockSpec((BLKH, 128), lambda i: (i, 0)),
        pl.BlockSpec((2, 2, 128, 128), lambda i: (0, 0, 0, 0)),
        pl.BlockSpec((2, 128), lambda i: (0, 0)),
    ],
    out_specs=pl.BlockSpec((NC, BLKH, 128), lambda i: (0, i, 0)),
    out_shape=jax.ShapeDtypeStruct((NC, HROWS, 128), jnp.float32),
)


def _pool_body(acc_ref, dinv_ref, b_ref, bjs_ref, wc_ref, bc_ref,
               out_ref, pool_scr):
  i = pl.program_id(0)

  @pl.when(i == 0)
  def _():
    pool_scr[...] = jnp.zeros_like(pool_scr)

  d = dinv_ref[...]
  r0 = jnp.maximum(acc_ref[0] * d + b_ref[0:1, :], 0.0)
  r1 = jnp.maximum(acc_ref[1] * d + b_ref[1:2, :], 0.0)
  laneblk = jax.lax.broadcasted_iota(jnp.int32, (BLKH, 128), 1) // 32
  giota = jax.lax.broadcasted_iota(jnp.int32, (BLKH, N_GRAPHS), 1)
  for j in range(4):
    mj = laneblk == j
    zc = jnp.concatenate(
        [jnp.where(mj, r0, 0.0), jnp.where(mj, r1, 0.0),
         jnp.ones((BLKH, 8), jnp.float32)], axis=1)
    ohj = (bjs_ref[:, j:j + 1] == giota).astype(jnp.float32)
    pool_scr[...] += jax.lax.dot_general(
        ohj, zc, (((0,), (0,)), ((), ())),
        precision=_HIGH, preferred_element_type=jnp.float32)

  @pl.when(i == GRID - 1)
  def _():
    pool = pool_scr[...]
    l32 = jax.lax.broadcasted_iota(jnp.int32, (128, 32), 0) % 32
    fold = (l32 == jax.lax.broadcasted_iota(jnp.int32, (128, 32), 1)
            ).astype(jnp.float32)
    sums = jnp.concatenate(
        [_dot(pool[:, :128], fold), _dot(pool[:, 128:256], fold)], axis=1)
    counts = jnp.maximum(pool[:, 256:257], 1.0)
    out_ref[...] = _dot(sums / counts, wc_ref[...]) + bc_ref[...]


_pool = pl.pallas_call(
    _pool_body,
    grid=(GRID,),
    in_specs=[
        pl.BlockSpec((NC, BLKH, 128), lambda i: (0, i, 0)),
        pl.BlockSpec((BLKH, 128), lambda i: (i, 0)),
        pl.BlockSpec((2, 128), lambda i: (0, 0)),
        pl.BlockSpec((BLKH, 4), lambda i: (i, 0)),
        pl.BlockSpec((HID, N_CLASSES), lambda i: (0, 0)),
        pl.BlockSpec((1, N_CLASSES), lambda i: (0, 0)),
    ],
    out_specs=pl.BlockSpec((N_GRAPHS, N_CLASSES), lambda i: (0, 0)),
    out_shape=jax.ShapeDtypeStruct((N_GRAPHS, N_CLASSES), jnp.float32),
    scratch_shapes=[pltpu.VMEM((N_GRAPHS, 264), jnp.float32)],
)


def _bd(W):
  eye4 = jnp.eye(4, dtype=W.dtype)
  return jnp.stack([
      jnp.stack([jnp.kron(eye4, W[i * 32:(i + 1) * 32, o * 32:(o + 1) * 32])
                 for o in range(2)])
      for i in range(2)])


def _bt(b):
  return jnp.stack([jnp.tile(b[:32], 4), jnp.tile(b[32:], 4)])


@jax.jit
def kernel(x, edge_index, batch, W1, b1, W2, b2, W3, b3, Wc, bc):
  row = edge_index[0].astype(jnp.int32)
  col = edge_index[1].astype(jnp.int32)
  npad = E_PAD - N_EDGES
  junkc = N_NODES + (jnp.arange(npad, dtype=jnp.int32) % (NPAD - N_NODES))
  row2d = jnp.concatenate([row, jnp.zeros((npad,), jnp.int32)]).reshape(
      N_CHUNKS, CHUNK)
  col2d = jnp.concatenate([col, junkc]).reshape(N_CHUNKS, CHUNK)
  bjs = jnp.concatenate(
      [batch.astype(jnp.int32),
       jnp.full((NPAD - N_NODES,), -1, jnp.int32)]).reshape(HROWS, 4)

  e0 = jnp.zeros((CHUNK, 16), jnp.float32).at[:, 0].set(1.0)
  zerosD = jnp.zeros((NPAD, 16), jnp.float32)

  def pack(a):
    return a.reshape(NC, HROWS, 128)

  def unpack(a):
    return a.reshape(NC, NPAD, HID // 2)

  deg = _deg_call(col2d, e0, zerosD)
  h1 = _mm1(x, W1)
  hs1, dinv32 = _scale1(h1, deg)
  dinvP = dinv32.reshape(HROWS, 128)
  acc1 = _agg_call(hs1, row2d, col2d)
  hs2 = _layer(pack(acc1), dinvP, _bd(W2), _bt(b1))
  acc2 = _agg_call(unpack(hs2), row2d, col2d)
  hs3 = _layer(pack(acc2), dinvP, _bd(W3), _bt(b2))
  acc3 = _agg_call(unpack(hs3), row2d, col2d)
  return _pool(pack(acc3), dinvP, _bt(b3), bjs,
               Wc, bc.reshape(1, N_CLASSES))
